# Initial kernel scaffold; baseline (speedup 1.0000x reference)
import functools

import jax
import jax.numpy as jnp
from jax import lax
from jax.experimental import pallas as pl
from jax.experimental.pallas import tpu as pltpu
from jax.experimental.pallas import tpu_sc as plsc

_G = 128


@functools.lru_cache(maxsize=None)
def _build_gather(B, V, D):
    info = plsc.get_sparse_core_info()
    nc, ns = info.num_cores, info.num_subcores
    nw = nc * ns
    rows_per_w = B // nw
    n_steps = rows_per_w // _G
    mesh = plsc.VectorSubcoreMesh(core_axis_name="c", subcore_axis_name="s")

    @functools.partial(
        pl.kernel,
        mesh=mesh,
        out_type=jax.ShapeDtypeStruct((B, D), jnp.float32),
        scratch_types=[
            pltpu.VMEM((n_steps, _G), jnp.int32),
            pltpu.VMEM((_G, D), jnp.float32),
            pltpu.SemaphoreType.DMA,
        ],
    )
    def k(idx_hbm, table_hbm, out_hbm, idx_v, rows_v, sem):
        wid = lax.axis_index("s") * nc + lax.axis_index("c")
        row0 = wid * n_steps
        pltpu.sync_copy(idx_hbm.at[pl.ds(row0, n_steps)], idx_v)

        def step(j, carry):
            pltpu.async_copy(table_hbm.at[idx_v.at[j]], rows_v, sem).wait()
            pltpu.sync_copy(rows_v, out_hbm.at[pl.ds((row0 + j) * _G, _G)])
            return carry

        lax.fori_loop(0, n_steps, step, 0)

    return k


def kernel(inputs, embeddings):
    B0, S = inputs.shape
    V, D = embeddings.shape
    B = B0 * S
    idx = inputs.reshape(B // _G, _G).astype(jnp.int32)
    out = _build_gather(B, V, D)(idx, embeddings)
    return out.reshape(B0, S, D)


# speedup vs baseline: 4.0833x; 4.0833x over previous
import functools

import jax
import jax.numpy as jnp
from jax import lax
from jax.experimental import pallas as pl
from jax.experimental.pallas import tpu as pltpu
from jax.experimental.pallas import tpu_sc as plsc

_G = 128


@functools.lru_cache(maxsize=None)
def _build_gather(B, V, D):
    info = plsc.get_sparse_core_info()
    nc, ns = info.num_cores, info.num_subcores
    nw = nc * ns
    rows_per_w = B // nw
    n_steps = rows_per_w // _G
    mesh = plsc.VectorSubcoreMesh(core_axis_name="c", subcore_axis_name="s")

    @functools.partial(
        pl.kernel,
        mesh=mesh,
        out_type=jax.ShapeDtypeStruct((B, D), jnp.float32),
        scratch_types=[
            pltpu.VMEM((rows_per_w,), jnp.int32),
            pltpu.VMEM((_G, D), jnp.float32),
            pltpu.SemaphoreType.DMA,
        ],
        compiler_params=pltpu.CompilerParams(use_tc_tiling_on_sc=False),
    )
    def k(idx_hbm, table_hbm, out_hbm, idx_v, rows_v, sem):
        wid = lax.axis_index("s") * nc + lax.axis_index("c")
        base = wid * rows_per_w
        pltpu.sync_copy(idx_hbm.at[pl.ds(base, rows_per_w)], idx_v)

        def step(j, carry):
            pltpu.async_copy(
                table_hbm.at[idx_v.at[pl.ds(j * _G, _G)]], rows_v, sem
            ).wait()
            pltpu.sync_copy(rows_v, out_hbm.at[pl.ds(base + j * _G, _G)])
            return carry

        lax.fori_loop(0, n_steps, step, 0)

    return k


def kernel(inputs, embeddings):
    B0, S = inputs.shape
    V, D = embeddings.shape
    B = B0 * S
    idx = inputs.reshape(B).astype(jnp.int32)
    out = _build_gather(B, V, D)(idx, embeddings)
    return out.reshape(B0, S, D)


# speedup vs baseline: 4.6907x; 1.1488x over previous
import functools

import jax
import jax.numpy as jnp
from jax import lax
from jax.experimental import pallas as pl
from jax.experimental.pallas import tpu as pltpu
from jax.experimental.pallas import tpu_sc as plsc

_G = 128
_NBUF = 4


@functools.lru_cache(maxsize=None)
def _build_gather(B, V, D):
    info = plsc.get_sparse_core_info()
    nc, ns = info.num_cores, info.num_subcores
    nw = nc * ns
    rows_per_w = B // nw
    n = rows_per_w // _G
    gmax = (n - 2 * _NBUF + 1) // _NBUF
    main_end = _NBUF * (gmax + 1)
    assert gmax >= 1 and main_end <= n
    mesh = plsc.VectorSubcoreMesh(core_axis_name="c", subcore_axis_name="s")

    @functools.partial(
        pl.kernel,
        mesh=mesh,
        out_type=jax.ShapeDtypeStruct((B, D), jnp.float32),
        scratch_types=[
            pltpu.VMEM((rows_per_w,), jnp.int32),
            [pltpu.VMEM((_G, D), jnp.float32) for _ in range(_NBUF)],
            pltpu.SemaphoreType.DMA,
            pltpu.SemaphoreType.DMA,
        ],
        compiler_params=pltpu.CompilerParams(use_tc_tiling_on_sc=False),
    )
    def k(idx_hbm, table_hbm, out_hbm, idx_v, bufs, sem_g, sem_o):
        wid = lax.axis_index("s") * nc + lax.axis_index("c")
        base = wid * rows_per_w
        pltpu.sync_copy(idx_hbm.at[pl.ds(base, rows_per_w)], idx_v)

        def gather(j, buf):
            pltpu.async_copy(
                table_hbm.at[idx_v.at[pl.ds(j * _G, _G)]], buf, sem_g
            )

        def put(j, buf):
            pltpu.async_copy(buf, out_hbm.at[pl.ds(base + j * _G, _G)], sem_o)

        def wait_gather(buf):
            pltpu.make_async_copy(table_hbm.at[pl.ds(0, _G)], buf, sem_g).wait()

        def wait_out(buf):
            pltpu.make_async_copy(buf, out_hbm.at[pl.ds(0, _G)], sem_o).wait()

        def step(j, b, first=False):
            wait_gather(bufs[b])
            put(j, bufs[b])
            nb = (b + _NBUF - 1) % _NBUF
            if not first:
                wait_out(bufs[nb])
            gather(j + _NBUF - 1, bufs[nb])

        for j in range(_NBUF - 1):
            gather(j, bufs[j])

        for j in range(_NBUF):
            step(j, j % _NBUF, first=(j == 0))

        def group(g, carry):
            for b in range(_NBUF):
                step(g * _NBUF + b, b)
            return carry

        lax.fori_loop(1, gmax + 1, group, 0)

        for j in range(main_end, n):
            b = j % _NBUF
            wait_gather(bufs[b])
            put(j, bufs[b])
            if j + _NBUF - 1 < n:
                nb = (b + _NBUF - 1) % _NBUF
                wait_out(bufs[nb])
                gather(j + _NBUF - 1, bufs[nb])

        for i in range(_NBUF):
            wait_out(bufs[i])

    return k


def kernel(inputs, embeddings):
    B0, S = inputs.shape
    V, D = embeddings.shape
    B = B0 * S
    idx = inputs.reshape(B).astype(jnp.int32)
    out = _build_gather(B, V, D)(idx, embeddings)
    return out.reshape(B0, S, D)
